# Initial kernel scaffold; baseline (speedup 1.0000x reference)
import functools

import jax
import jax.numpy as jnp
from jax.experimental import pallas as pl
from jax.experimental.pallas import tpu as pltpu


def _moe_kernel(xb_ref, wr_ref, wge_ref, wue_ref, wde_ref, wgs_ref, wus_ref,
                wds_ref, out_ref, combine_ref, *, tt_chunk):
    e = pl.program_id(0)
    f = pl.program_id(1)
    T = xb_ref.shape[0]
    E = wr_ref.shape[0]

    @pl.when((e == 0) & (f == 0))
    def _router():
        logits = jax.lax.dot_general(
            xb_ref[...], wr_ref[...], (((1,), (1,)), ((), ())),
            preferred_element_type=jnp.float32)
        m1 = jnp.max(logits, axis=1, keepdims=True)
        eq1 = (logits == m1)
        fo1 = eq1 & (jnp.cumsum(eq1.astype(jnp.int32), axis=1) == 1)
        masked = jnp.where(fo1, -1e30, logits)
        m2 = jnp.max(masked, axis=1, keepdims=True)
        eq2 = (masked == m2)
        fo2 = eq2 & (jnp.cumsum(eq2.astype(jnp.int32), axis=1) == 1)
        combine = (jax.nn.sigmoid(m1) * fo1.astype(jnp.float32)
                   + jax.nn.sigmoid(m2) * fo2.astype(jnp.float32))
        combine_ref[...] = combine
        out_ref[...] = jnp.zeros_like(out_ref)

    is_shared = (e == E).astype(jnp.float32)
    lane = jax.lax.broadcasted_iota(jnp.int32, (T, E), 1)
    col = jnp.sum(jnp.where(lane == e, combine_ref[...], 0.0), axis=1,
                  keepdims=True)
    col = col + is_shared

    wg = (wge_ref[0] * (1.0 - is_shared) + wgs_ref[...] * is_shared)
    wu = (wue_ref[0] * (1.0 - is_shared) + wus_ref[...] * is_shared)
    wd = (wde_ref[0] * (1.0 - is_shared) + wds_ref[...] * is_shared)
    wg = wg.astype(jnp.bfloat16)
    wu = wu.astype(jnp.bfloat16)
    wd = wd.astype(jnp.bfloat16)

    for tt in range(T // tt_chunk):
        sl = pl.ds(tt * tt_chunk, tt_chunk)
        xc = xb_ref[sl, :]
        gate = jax.lax.dot_general(xc, wg, (((1,), (1,)), ((), ())),
                                   preferred_element_type=jnp.float32)
        up = jax.lax.dot_general(xc, wu, (((1,), (1,)), ((), ())),
                                 preferred_element_type=jnp.float32)
        act = gate * jax.nn.sigmoid(gate) * up
        acts = (act * col[sl, :]).astype(jnp.bfloat16)
        part = jax.lax.dot_general(acts, wd, (((1,), (1,)), ((), ())),
                                   preferred_element_type=jnp.float32)
        out_ref[sl, :] += part


def kernel(hidden_states, W_router, Wg_experts, Wu_experts, Wd_experts,
           Wg_shared, Wu_shared, Wd_shared):
    T, H = hidden_states.shape
    E, F, _ = Wg_experts.shape
    BF = min(256, F)
    n_f = F // BF
    tt_chunk = max(T // 4, 8)

    xb = hidden_states.astype(jnp.bfloat16)
    wrb = W_router.astype(jnp.bfloat16)

    grid = (E + 1, n_f)
    kfn = functools.partial(_moe_kernel, tt_chunk=tt_chunk)
    out = pl.pallas_call(
        kfn,
        grid=grid,
        in_specs=[
            pl.BlockSpec((T, H), lambda e, f: (0, 0)),
            pl.BlockSpec((E, H), lambda e, f: (0, 0)),
            pl.BlockSpec((1, BF, H), lambda e, f: (jnp.minimum(e, E - 1), f, 0)),
            pl.BlockSpec((1, BF, H), lambda e, f: (jnp.minimum(e, E - 1), f, 0)),
            pl.BlockSpec((1, H, BF), lambda e, f: (jnp.minimum(e, E - 1), 0, f)),
            pl.BlockSpec((BF, H), lambda e, f: (jnp.where(e == E, f, 0), 0)),
            pl.BlockSpec((BF, H), lambda e, f: (jnp.where(e == E, f, 0), 0)),
            pl.BlockSpec((H, BF), lambda e, f: (0, jnp.where(e == E, f, 0))),
        ],
        out_specs=pl.BlockSpec((T, H), lambda e, f: (0, 0)),
        out_shape=jax.ShapeDtypeStruct((T, H), jnp.float32),
        scratch_shapes=[pltpu.VMEM((T, E), jnp.float32)],
        compiler_params=pltpu.CompilerParams(
            dimension_semantics=("arbitrary", "arbitrary")),
    )(xb, wrb, Wg_experts, Wu_experts, Wd_experts, Wg_shared, Wu_shared,
      Wd_shared)
    return out


# speedup vs baseline: 1.5188x; 1.5188x over previous
import functools

import jax
import jax.numpy as jnp
from jax.experimental import pallas as pl
from jax.experimental.pallas import tpu as pltpu


def _moe_kernel(xb_ref, wr_ref, wge_ref, wue_ref, wde_ref, wgs_ref, wus_ref,
                wds_ref, out_ref, combine_ref, *, tt_chunk):
    e = pl.program_id(0)
    f = pl.program_id(1)
    T = xb_ref.shape[0]
    E = wr_ref.shape[0]

    @pl.when((e == 0) & (f == 0))
    def _router():
        logits = jax.lax.dot_general(
            xb_ref[...], wr_ref[...], (((1,), (1,)), ((), ())),
            preferred_element_type=jnp.float32)
        ii = jax.lax.broadcasted_iota(jnp.int32, (E, E), 0)
        jj = jax.lax.broadcasted_iota(jnp.int32, (E, E), 1)
        tri = (ii <= jj).astype(jnp.float32)

        def first_occurrence(eq):
            cums = jax.lax.dot_general(eq.astype(jnp.float32), tri,
                                       (((1,), (0,)), ((), ())),
                                       preferred_element_type=jnp.float32)
            return eq & (cums == 1.0)

        m1 = jnp.max(logits, axis=1, keepdims=True)
        fo1 = first_occurrence(logits == m1)
        masked = jnp.where(fo1, -1e30, logits)
        m2 = jnp.max(masked, axis=1, keepdims=True)
        fo2 = first_occurrence(masked == m2)
        combine = (jax.nn.sigmoid(m1) * fo1.astype(jnp.float32)
                   + jax.nn.sigmoid(m2) * fo2.astype(jnp.float32))
        combine_ref[...] = combine
        out_ref[...] = jnp.zeros_like(out_ref)

    is_shared = (e == E).astype(jnp.float32)
    lane = jax.lax.broadcasted_iota(jnp.int32, (T, E), 1)
    col = jnp.sum(jnp.where(lane == e, combine_ref[...], 0.0), axis=1,
                  keepdims=True)
    col = col + is_shared

    wg = (wge_ref[0] * (1.0 - is_shared) + wgs_ref[...] * is_shared)
    wu = (wue_ref[0] * (1.0 - is_shared) + wus_ref[...] * is_shared)
    wd = (wde_ref[0] * (1.0 - is_shared) + wds_ref[...] * is_shared)
    wg = wg.astype(jnp.bfloat16)
    wu = wu.astype(jnp.bfloat16)
    wd = wd.astype(jnp.bfloat16)

    for tt in range(T // tt_chunk):
        sl = slice(tt * tt_chunk, (tt + 1) * tt_chunk)
        xc = xb_ref[sl, :]
        gate = jax.lax.dot_general(xc, wg, (((1,), (1,)), ((), ())),
                                   preferred_element_type=jnp.float32)
        up = jax.lax.dot_general(xc, wu, (((1,), (1,)), ((), ())),
                                 preferred_element_type=jnp.float32)
        act = gate * jax.nn.sigmoid(gate) * up
        acts = (act * col[sl, :]).astype(jnp.bfloat16)
        part = jax.lax.dot_general(acts, wd, (((1,), (1,)), ((), ())),
                                   preferred_element_type=jnp.float32)
        out_ref[sl, :] += part


def kernel(hidden_states, W_router, Wg_experts, Wu_experts, Wd_experts,
           Wg_shared, Wu_shared, Wd_shared):
    T, H = hidden_states.shape
    E, F, _ = Wg_experts.shape
    BF = min(256, F)
    n_f = F // BF
    tt_chunk = max(T // 4, 8)

    xb = hidden_states.astype(jnp.bfloat16)
    wrb = W_router.astype(jnp.bfloat16)

    grid = (E + 1, n_f)
    kfn = functools.partial(_moe_kernel, tt_chunk=tt_chunk)
    out = pl.pallas_call(
        kfn,
        grid=grid,
        in_specs=[
            pl.BlockSpec((T, H), lambda e, f: (0, 0)),
            pl.BlockSpec((E, H), lambda e, f: (0, 0)),
            pl.BlockSpec((1, BF, H), lambda e, f: (jnp.minimum(e, E - 1), f, 0)),
            pl.BlockSpec((1, BF, H), lambda e, f: (jnp.minimum(e, E - 1), f, 0)),
            pl.BlockSpec((1, H, BF), lambda e, f: (jnp.minimum(e, E - 1), 0, f)),
            pl.BlockSpec((BF, H), lambda e, f: (jnp.where(e == E, f, 0), 0)),
            pl.BlockSpec((BF, H), lambda e, f: (jnp.where(e == E, f, 0), 0)),
            pl.BlockSpec((H, BF), lambda e, f: (0, jnp.where(e == E, f, 0))),
        ],
        out_specs=pl.BlockSpec((T, H), lambda e, f: (0, 0)),
        out_shape=jax.ShapeDtypeStruct((T, H), jnp.float32),
        scratch_shapes=[pltpu.VMEM((T, E), jnp.float32)],
        compiler_params=pltpu.CompilerParams(
            dimension_semantics=("arbitrary", "arbitrary")),
    )(xb, wrb, Wg_experts, Wu_experts, Wd_experts, Wg_shared, Wu_shared,
      Wd_shared)
    return out
